# baseline (speedup 1.0000x reference)
import contextlib
import functools
import math

import numpy as np
import jax
import jax.numpy as jnp
from jax.experimental import pallas as pl

_N = 10000
_E = 320000
_D = 128
_H = 128
_C = 64
_K = 10
_ALPHA = 0.1
_EC = 0.5
_M = 2 * _E + _N


def _cpu_ctx():
    try:
        return jax.default_device(jax.devices("cpu")[0])
    except Exception:
        return contextlib.nullcontext()


def _precompute():
    att_np = _ALPHA * (1 - _ALPHA) ** np.arange(_K + 1, dtype=np.float64)
    att_np[-1] = (1 - _ALPHA) ** _K
    num_edgess = int(_N * math.log(1.0 * _N) * _EC)
    with _cpu_ctx():
        att = jnp.asarray(att_np, jnp.float32)
        key = jax.random.key(42)
        key, sk = jax.random.split(key)
        samp_len = jax.random.categorical(sk, jnp.log(att), shape=(num_edgess,))
        samp_len = np.asarray(samp_len)
        counts = [int(np.sum(samp_len == i)) for i in range(1, _K + 1)]
        idxs, sls, srs, uls, urs = [], [], [], [], []
        for i in range(1, _K + 1):
            ne = counts[i - 1]
            if ne == 0:
                continue
            key, k1, k2, k3, k4 = jax.random.split(key, 5)
            idx = np.asarray(jax.random.randint(k1, (ne,), 0, _M))
            idl = np.asarray(jax.random.randint(k2, (ne,), 0, i))
            idr = (i - 1) - idl
            ul = np.zeros((_K - 1, ne), np.float32)
            ur = np.zeros((_K - 1, ne), np.float32)
            kk = k3
            for t in range(i - 1):
                kk, skk = jax.random.split(kk)
                ul[t] = np.asarray(jax.random.uniform(skk, (ne,)))
            kk = k4
            for t in range(i - 1):
                kk, skk = jax.random.split(kk)
                ur[t] = np.asarray(jax.random.uniform(skk, (ne,)))
            idxs.append(idx)
            sls.append(idl)
            srs.append(idr)
            uls.append(ul)
            urs.append(ur)
    idx = np.concatenate(idxs).astype(np.int32)
    sl = np.concatenate(sls).astype(np.int32)
    sr = np.concatenate(srs).astype(np.int32)
    ul = np.concatenate(uls, axis=1)
    ur = np.concatenate(urs, axis=1)
    scale = np.float32(_M / num_edgess)
    return idx, sl, sr, ul, ur, scale


_IDX, _SL, _SR, _UL, _UR, _SCALE = _precompute()
_S = _IDX.shape[0]
_ATT0 = np.float32(_ALPHA)


def _mlp_body(x_ref, w1_ref, b1_ref, w2_ref, b2_ref, out_ref):
    h1 = jnp.dot(x_ref[...], w1_ref[...], preferred_element_type=jnp.float32)
    h1 = jnp.maximum(h1 + b1_ref[...], 0.0)
    out_ref[...] = (
        jnp.dot(h1, w2_ref[...], preferred_element_type=jnp.float32) + b2_ref[...]
    )


def _mlp(x, W1, b1, W2, b2):
    blk = 2000
    grid = _N // blk
    return pl.pallas_call(
        _mlp_body,
        grid=(grid,),
        in_specs=[
            pl.BlockSpec((blk, _D), lambda i: (i, 0)),
            pl.BlockSpec((_D, _H), lambda i: (0, 0)),
            pl.BlockSpec((1, _H), lambda i: (0, 0)),
            pl.BlockSpec((_H, _C), lambda i: (0, 0)),
            pl.BlockSpec((1, _C), lambda i: (0, 0)),
        ],
        out_specs=pl.BlockSpec((blk, _C), lambda i: (i, 0)),
        out_shape=jax.ShapeDtypeStruct((_N, _C), jnp.float32),
    )(x, W1, b1.reshape(1, _H), W2, b2.reshape(1, _C))


def _final_body(h_ref, p_ref, out_ref):
    z = h_ref[...] * _ATT0 + p_ref[...]
    m = jnp.max(z, axis=1, keepdims=True)
    e = jnp.exp(z - m)
    s = jnp.sum(e, axis=1, keepdims=True)
    out_ref[...] = (z - m) - jnp.log(s)


def _final(h, prop):
    blk = 2000
    grid = _N // blk
    return pl.pallas_call(
        _final_body,
        grid=(grid,),
        in_specs=[
            pl.BlockSpec((blk, _C), lambda i: (i, 0)),
            pl.BlockSpec((blk, _C), lambda i: (i, 0)),
        ],
        out_specs=pl.BlockSpec((blk, _C), lambda i: (i, 0)),
        out_shape=jax.ShapeDtypeStruct((_N, _C), jnp.float32),
    )(h, prop)


def kernel(x, edge_index, W1, b1, W2, b2):
    src, dst = edge_index[0], edge_index[1]
    loops = jnp.arange(_N, dtype=src.dtype)
    row = jnp.concatenate([src, dst, loops])
    col = jnp.concatenate([dst, src, loops])
    order = jnp.argsort(row)
    row_s = row[order]
    col_s = col[order]
    degree = jnp.bincount(row, length=_N)
    rowptr = jnp.concatenate([jnp.zeros((1,), degree.dtype), jnp.cumsum(degree)])
    degf = degree.astype(jnp.float32)

    h = _mlp(x, W1, b1, W2, b2)

    idx = jnp.asarray(_IDX)
    cur_l = row_s[idx]
    cur_r = col_s[idx]
    sl = jnp.asarray(_SL)
    sr = jnp.asarray(_SR)

    def _walk(cur, steps, u_all):
        for t in range(_K - 1):
            active = t < steps
            d = degree[cur]
            u = jnp.asarray(u_all[t])
            off = jnp.minimum(jnp.floor(u * degf[cur]).astype(d.dtype), d - 1)
            nxt = col_s[rowptr[cur] + off]
            cur = jnp.where(active, nxt, cur)
        return cur

    le = _walk(cur_l, sl, _UL)
    re = _walk(cur_r, sr, _UR)
    val = degf[le] ** -0.5 * degf[re] ** -0.5 * _SCALE

    prop = (
        jnp.zeros((_N, _C), jnp.float32)
        .at[le]
        .add(val[:, None] * h[re])
        .at[re]
        .add(val[:, None] * h[le])
    )
    return _final(h, prop)
